# baseline (speedup 1.0000x reference)
import functools

import jax
import jax.numpy as jnp
from jax import lax
from jax.experimental import pallas as pl
from jax.experimental.pallas import tpu as pltpu
from jax.experimental.pallas import tpu_sc as plsc

B, C, H, W = 4, 64, 192, 192
N = 5000
OUT = 7
NW = 32
NPER = 160
NPAD = NW * NPER
PAIR_ROWS = B * H * W // 2
CROW = C * OUT * OUT


def _taps_axis(start, binsz, base_i):
  pv = lax.iota(jnp.int32, 16).astype(jnp.float32)
  x0 = start + (pv + 0.25) * binsz
  x1 = start + (pv + 0.75) * binsz
  l0 = x0.astype(jnp.int32)
  l1 = x1.astype(jnp.int32)
  lx0 = x0 - l0.astype(jnp.float32)
  hx0 = 1.0 - lx0
  lx1 = x1 - l1.astype(jnp.float32)
  hx1 = 1.0 - lx1
  deq0 = (l1 - l0) == 0
  z = jnp.zeros_like(lx1)
  w0 = hx0 + jnp.where(deq0, hx1, z)
  w1 = lx0 + jnp.where(deq0, lx1, hx1)
  w2 = jnp.where(deq0, z, lx1)
  jv = l0 - base_i
  return w0, w1, w2, jv


def _body(feat_hbm, bbt_hbm, out_hbm, bbv, idx0, patch0, stag0,
          idx1, patch1, stag1,
          s_sw, s_bw, s_sh, s_bh, s_xb, s_yb, s_base,
          w_y0, w_y1, w_y2, w_jy,
          gsem0, gsem1, osem0, osem1):
  wid = lax.axis_index("s") * 2 + lax.axis_index("c")
  t0 = wid * NPER
  nloc = jnp.clip(N - t0, 0, NPER)

  for f in range(5):
    pltpu.sync_copy(bbt_hbm.at[pl.ds(f * NPAD + t0, NPER)],
                    bbv.at[pl.ds(f * NPER, NPER)])

  lane = lax.iota(jnp.int32, 16)
  bufs = ((idx0, patch0, gsem0), (idx1, patch1, gsem1))
  stags = ((stag0, osem0), (stag1, osem1))

  def gen_issue(ir, bufset):
    idx_r, patch_r, gsem = bufset
    base = s_base[ir]
    patt = jnp.where(lane < 8, lane, lane + (W // 2 - 8))

    def gen(v, _):
      idx_r[pl.ds(v * 16, 16)] = patt + (base + v * W)
      return 0

    lax.fori_loop(0, 8, gen, 0)
    pltpu.make_async_copy(feat_hbm.at[idx_r], patch_r, gsem).start()

  def compute(ir, i, bufset, stag_r, off8):
    idx_r, patch_r, gsem = bufset
    pltpu.make_async_copy(feat_hbm.at[idx_r], patch_r, gsem).wait()

    wx0, wx1, wx2, jxv = _taps_axis(s_sw[ir], s_bw[ir], s_xb[ir])
    wy0, wy1, wy2, jyv = _taps_axis(s_sh[ir], s_bh[ir], s_yb[ir])
    for p in range(OUT):
      w_y0[p] = wy0[p]
      w_y1[p] = wy1[p]
      w_y2[p] = wy2[p]
      w_jy[p] = jyv[p]

    xw = [(wx0[p], wx1[p], wx2[p]) for p in range(OUT)]
    xj = []
    for p in range(OUT):
      jx = jxv[p]
      jxh = jx >> 1
      jxp = jnp.bitwise_and(jx, 1)
      a0 = jxp * 64
      xj.append((jxh, jxp, a0, 64 - a0))

    lane49 = lane * 49 + off8

    def ph_loop(ph, _):
      wy = (w_y0[ph], w_y1[ph], w_y2[ph])
      jy = w_jy[ph]
      row_i = (jy * 8, (jy + 1) * 8, (jy + 2) * 8)
      for pw in range(OUT):
        jxh, jxp, a0, a1 = xj[pw]
        accs = [jnp.zeros((16,), jnp.float32) for _ in range(4)]
        for ii in range(3):
          rows = (row_i[ii] + jxh, row_i[ii] + jxh + jxp, row_i[ii] + jxh + 1)
          coffs = (a0, a1, a0)
          for jj in range(3):
            wij = wy[ii] * xw[pw][jj]
            for cch in range(4):
              accs[cch] = accs[cch] + wij * patch_r[
                  rows[jj], pl.ds(coffs[jj] + cch * 16, 16)]
        k = ph * OUT + pw
        for cch in range(4):
          plsc.store_scatter(stag_r, [lane49 + (cch * 784 + k)],
                             accs[cch] * 0.25)
      return 0

    lax.fori_loop(0, OUT, ph_loop, 0)

  def grp(iq, _):
    o = iq * 16
    bv = bbv[pl.ds(o, 16)]
    swv = bbv[pl.ds(NPER + o, 16)]
    shv = bbv[pl.ds(2 * NPER + o, 16)]
    ewv = bbv[pl.ds(3 * NPER + o, 16)]
    ehv = bbv[pl.ds(4 * NPER + o, 16)]
    bwv = jnp.maximum(ewv - swv, 1.0) * (1.0 / OUT)
    bhv = jnp.maximum(ehv - shv, 1.0) * (1.0 / OUT)
    xbv = jnp.bitwise_and(
        jnp.clip((swv + 0.25 * bwv).astype(jnp.int32), 0, W - 16), -2)
    ybv = jnp.clip((shv + 0.25 * bhv).astype(jnp.int32), 0, H - 16)
    basev = ((bv.astype(jnp.int32) * H + ybv) * W + xbv) >> 1
    for ir in range(16):
      s_sw[ir] = swv[ir]
      s_bw[ir] = bwv[ir]
      s_sh[ir] = shv[ir]
      s_bh[ir] = bhv[ir]
      s_xb[ir] = xbv[ir]
      s_yb[ir] = ybv[ir]
      s_base[ir] = basev[ir]

    @pl.when(o < nloc)
    def _():
      gen_issue(0, bufs[0])

    for half in range(2):
      stag_r, osem = stags[half]
      gb = iq * 2 + half

      @pl.when(o + half * 8 < nloc)
      def _(half=half, stag_r=stag_r, osem=osem, gb=gb):
        @pl.when(gb >= 2)
        def _():
          prow = wid * (NPER // 8) + gb - 2
          pltpu.make_async_copy(stag_r, out_hbm.at[prow], osem).wait()

        def pair_loop(p, _2):
          for b in range(2):
            ir = half * 8 + 2 * p + b
            i = o + ir

            @pl.when(jnp.logical_and(ir < 15, i + 1 < nloc))
            def _(ir=ir, b=b):
              gen_issue(ir + 1, bufs[1 - b])

            compute(ir, i, bufs[b], stag_r, (2 * p + b) * CROW)

          return 0

        lax.fori_loop(0, 4, pair_loop, 0)
        pltpu.make_async_copy(
            stag_r, out_hbm.at[wid * (NPER // 8) + gb], osem).start()

    return 0

  lax.fori_loop(0, NPER // 16, grp, 0)

  nb = nloc >> 3
  m = nb - 1
  rbase = wid * (NPER // 8)

  @pl.when(jnp.bitwise_and(m, 1) == 0)
  def _():
    pltpu.make_async_copy(stag0, out_hbm.at[rbase + m], osem0).wait()
    pltpu.make_async_copy(stag1, out_hbm.at[rbase + m - 1], osem1).wait()

  @pl.when(jnp.bitwise_and(m, 1) == 1)
  def _():
    pltpu.make_async_copy(stag1, out_hbm.at[rbase + m], osem1).wait()
    pltpu.make_async_copy(stag0, out_hbm.at[rbase + m - 1], osem0).wait()


_sc_call = functools.partial(
    pl.kernel,
    out_type=jax.ShapeDtypeStruct((N // 8, 8 * CROW), jnp.float32),
    mesh=plsc.VectorSubcoreMesh(core_axis_name="c", subcore_axis_name="s"),
    compiler_params=pltpu.CompilerParams(needs_layout_passes=False),
    scratch_types=[
        pltpu.VMEM((5 * NPER,), jnp.float32),
        pltpu.VMEM((128,), jnp.int32),
        pltpu.VMEM((128, 128), jnp.float32),
        pltpu.VMEM((8 * CROW,), jnp.float32),
        pltpu.VMEM((128,), jnp.int32),
        pltpu.VMEM((128, 128), jnp.float32),
        pltpu.VMEM((8 * CROW,), jnp.float32),
        pltpu.SMEM((16,), jnp.float32),
        pltpu.SMEM((16,), jnp.float32),
        pltpu.SMEM((16,), jnp.float32),
        pltpu.SMEM((16,), jnp.float32),
        pltpu.SMEM((16,), jnp.int32),
        pltpu.SMEM((16,), jnp.int32),
        pltpu.SMEM((16,), jnp.int32),
        pltpu.SMEM((8,), jnp.float32),
        pltpu.SMEM((8,), jnp.float32),
        pltpu.SMEM((8,), jnp.float32),
        pltpu.SMEM((8,), jnp.int32),
        pltpu.SemaphoreType.DMA,
        pltpu.SemaphoreType.DMA,
        pltpu.SemaphoreType.DMA,
        pltpu.SemaphoreType.DMA,
    ],
)(_body)


@jax.jit
def kernel(input, bboxes):
  feat = jnp.transpose(input.astype(jnp.float32), (0, 2, 3, 1))
  feat = feat.reshape(PAIR_ROWS, 2 * C)
  bbt = jnp.concatenate(
      [bboxes.astype(jnp.float32).T,
       jnp.zeros((5, NPAD - N), jnp.float32)], axis=1).reshape(5 * NPAD)
  out = _sc_call(feat, bbt)
  return out.reshape(N, C, OUT, OUT)


# speedup vs baseline: 6.8752x; 6.8752x over previous
import functools

import jax
import jax.numpy as jnp
from jax import lax
from jax.experimental import pallas as pl
from jax.experimental.pallas import tpu as pltpu
from jax.experimental.pallas import tpu_sc as plsc

B, C, H, W = 4, 64, 192, 192
N = 5000
OUT = 7
NW = 32
NPER = 160
NPAD = NW * NPER
PAIR_ROWS = B * H * W // 2
CROW = C * OUT * OUT


def _taps_axis(start, binsz, base_i, scale=1.0):
  pv = lax.iota(jnp.int32, 16).astype(jnp.float32)
  x0 = start + (pv + 0.25) * binsz
  x1 = start + (pv + 0.75) * binsz
  l0 = x0.astype(jnp.int32)
  l1 = x1.astype(jnp.int32)
  lx0 = x0 - l0.astype(jnp.float32)
  hx0 = 1.0 - lx0
  lx1 = x1 - l1.astype(jnp.float32)
  hx1 = 1.0 - lx1
  deq0 = (l1 - l0) == 0
  z = jnp.zeros_like(lx1)
  w0 = hx0 + jnp.where(deq0, hx1, z)
  w1 = lx0 + jnp.where(deq0, lx1, hx1)
  w2 = jnp.where(deq0, z, lx1)
  if scale != 1.0:
    w0 = w0 * scale
    w1 = w1 * scale
    w2 = w2 * scale
  jv = l0 - base_i
  return w0, w1, w2, jv


def _body(feat_hbm, bbt_hbm, out_hbm, bbv, idx0, patch0, stag0,
          idx1, patch1, stag1,
          s_sw, s_bw, s_sh, s_bh, s_xb, s_yb, s_base,
          w_y0, w_y1, w_y2, w_jy,
          gsem0, gsem1, osem0, osem1):
  wid = lax.axis_index("s") * 2 + lax.axis_index("c")
  t0 = wid * NPER
  nloc = jnp.clip(N - t0, 0, NPER)

  for f in range(5):
    pltpu.sync_copy(bbt_hbm.at[pl.ds(f * NPAD + t0, NPER)],
                    bbv.at[pl.ds(f * NPER, NPER)])

  lane = lax.iota(jnp.int32, 16)
  bufs = ((idx0, patch0, stag0, gsem0, osem0),
          (idx1, patch1, stag1, gsem1, osem1))

  def gen_issue(ir, bufset):
    idx_r, patch_r, _, gsem, _ = bufset
    base = s_base[ir]
    patt = jnp.where(lane < 8, lane, lane + (W // 2 - 8))

    for v in range(8):
      idx_r[pl.ds(v * 16, 16)] = patt + (base + v * W)
    pltpu.make_async_copy(feat_hbm.at[idx_r], patch_r, gsem).start()

  def compute(ir, i, bufset):
    idx_r, patch_r, stag_r, gsem, osem = bufset
    pltpu.make_async_copy(feat_hbm.at[idx_r], patch_r, gsem).wait()

    wx0, wx1, wx2, jxv = _taps_axis(s_sw[ir], s_bw[ir], s_xb[ir])
    wy0, wy1, wy2, jyv = _taps_axis(s_sh[ir], s_bh[ir], s_yb[ir], scale=0.25)
    for p in range(OUT):
      w_y0[p] = wy0[p]
      w_y1[p] = wy1[p]
      w_y2[p] = wy2[p]
      w_jy[p] = jyv[p]

    xw = [(wx0[p], wx1[p], wx2[p]) for p in range(OUT)]
    xj = []
    for p in range(OUT):
      jx = jxv[p]
      jxh = jx >> 1
      jxp = jnp.bitwise_and(jx, 1)
      a0 = jxp * 64
      xj.append((jxh, jxp, a0, 64 - a0))

    @pl.when(i >= 2)
    def _():
      pltpu.make_async_copy(stag_r, out_hbm.at[t0 + i - 2], osem).wait()

    lane49 = lane * 49

    def ph_loop(ph, _):
      wy = (w_y0[ph], w_y1[ph], w_y2[ph])
      jy = w_jy[ph]
      row_i = (jy * 8, (jy + 1) * 8, (jy + 2) * 8)
      for pw in range(OUT):
        jxh, jxp, a0, a1 = xj[pw]
        accs = [jnp.zeros((16,), jnp.float32) for _ in range(4)]
        for ii in range(3):
          rows = (row_i[ii] + jxh, row_i[ii] + jxh + jxp, row_i[ii] + jxh + 1)
          coffs = (a0, a1, a0)
          for jj in range(3):
            wij = wy[ii] * xw[pw][jj]
            for cch in range(4):
              accs[cch] = accs[cch] + wij * patch_r[
                  rows[jj], pl.ds(coffs[jj] + cch * 16, 16)]
        k = ph * OUT + pw
        for cch in range(4):
          plsc.store_scatter(stag_r, [lane49 + (cch * 784 + k)], accs[cch])
      return 0

    lax.fori_loop(0, OUT, ph_loop, 0)
    pltpu.make_async_copy(stag_r, out_hbm.at[t0 + i], osem).start()

  def grp(iq, _):
    o = iq * 16
    bv = bbv[pl.ds(o, 16)]
    swv = bbv[pl.ds(NPER + o, 16)]
    shv = bbv[pl.ds(2 * NPER + o, 16)]
    ewv = bbv[pl.ds(3 * NPER + o, 16)]
    ehv = bbv[pl.ds(4 * NPER + o, 16)]
    bwv = jnp.maximum(ewv - swv, 1.0) * (1.0 / OUT)
    bhv = jnp.maximum(ehv - shv, 1.0) * (1.0 / OUT)
    xbv = jnp.bitwise_and(
        jnp.clip((swv + 0.25 * bwv).astype(jnp.int32), 0, W - 16), -2)
    ybv = jnp.clip((shv + 0.25 * bhv).astype(jnp.int32), 0, H - 16)
    basev = ((bv.astype(jnp.int32) * H + ybv) * W + xbv) >> 1
    for ir in range(16):
      s_sw[ir] = swv[ir]
      s_bw[ir] = bwv[ir]
      s_sh[ir] = shv[ir]
      s_bh[ir] = bhv[ir]
      s_xb[ir] = xbv[ir]
      s_yb[ir] = ybv[ir]
      s_base[ir] = basev[ir]

    @pl.when(o < nloc)
    def _():
      gen_issue(0, bufs[0])

    def pair_loop(p, _2):
      for b in range(2):
        ir = 2 * p + b
        i = o + ir

        @pl.when(i < nloc)
        def _(ir=ir, i=i, b=b):
          @pl.when(jnp.logical_and(ir < 15, i + 1 < nloc))
          def _():
            gen_issue(ir + 1, bufs[1 - b])

          compute(ir, i, bufs[b])

      return 0

    lax.fori_loop(0, 8, pair_loop, 0)
    return 0

  lax.fori_loop(0, NPER // 16, grp, 0)

  pltpu.make_async_copy(stag0, out_hbm.at[t0 + nloc - 2], osem0).wait()
  pltpu.make_async_copy(stag1, out_hbm.at[t0 + nloc - 1], osem1).wait()


_sc_call = functools.partial(
    pl.kernel,
    out_type=jax.ShapeDtypeStruct((N, CROW), jnp.float32),
    mesh=plsc.VectorSubcoreMesh(core_axis_name="c", subcore_axis_name="s"),
    compiler_params=pltpu.CompilerParams(needs_layout_passes=False),
    scratch_types=[
        pltpu.VMEM((5 * NPER,), jnp.float32),
        pltpu.VMEM((128,), jnp.int32),
        pltpu.VMEM((128, 128), jnp.float32),
        pltpu.VMEM((CROW,), jnp.float32),
        pltpu.VMEM((128,), jnp.int32),
        pltpu.VMEM((128, 128), jnp.float32),
        pltpu.VMEM((CROW,), jnp.float32),
        pltpu.SMEM((16,), jnp.float32),
        pltpu.SMEM((16,), jnp.float32),
        pltpu.SMEM((16,), jnp.float32),
        pltpu.SMEM((16,), jnp.float32),
        pltpu.SMEM((16,), jnp.int32),
        pltpu.SMEM((16,), jnp.int32),
        pltpu.SMEM((16,), jnp.int32),
        pltpu.SMEM((8,), jnp.float32),
        pltpu.SMEM((8,), jnp.float32),
        pltpu.SMEM((8,), jnp.float32),
        pltpu.SMEM((8,), jnp.int32),
        pltpu.SemaphoreType.DMA,
        pltpu.SemaphoreType.DMA,
        pltpu.SemaphoreType.DMA,
        pltpu.SemaphoreType.DMA,
    ],
)(_body)


@jax.jit
def kernel(input, bboxes):
  feat = jnp.transpose(input.astype(jnp.float32), (0, 2, 3, 1))
  feat = feat.reshape(PAIR_ROWS, 2 * C)
  bbt = jnp.concatenate(
      [bboxes.astype(jnp.float32).T,
       jnp.zeros((5, NPAD - N), jnp.float32)], axis=1).reshape(5 * NPAD)
  out = _sc_call(feat, bbt)
  return out.reshape(N, C, OUT, OUT)
